# baseline (speedup 1.0000x reference)
import functools

import jax
import jax.numpy as jnp
from jax import lax
from jax.experimental import pallas as pl
from jax.experimental.pallas import tpu as pltpu
from jax.experimental.pallas import tpu_sc as plsc

_D = 64
_V = 1000000
_B = 16384
_L = 50
_NC, _NS = 2, 16
_NW = _NC * _NS
_BW = _B // _NW
_G = 256
_KSUB = _G // 128


_W = 16384
_GRID_T = 31
_H = _GRID_T * _W


def _tr_body(xlo_ref, xhi_ref, o_ref):
    o_ref[:, 0:_D] = xlo_ref[...].T
    o_ref[:, _D:2 * _D] = xhi_ref[...].T


def _transpose_table(lookup):
    return pl.pallas_call(
        _tr_body,
        grid=(_GRID_T,),
        in_specs=[
            pl.BlockSpec((_D, _W), lambda i: (0, i)),
            pl.BlockSpec((_D, _W), lambda i: (0, jnp.minimum(i + _GRID_T, 61))),
        ],
        out_specs=pl.BlockSpec((_W, 2 * _D), lambda i: (i, 0)),
        out_shape=jax.ShapeDtypeStruct((_H, 2 * _D), jnp.float32),
    )(lookup, lookup)


_MESH = plsc.VectorSubcoreMesh(core_axis_name="c", subcore_axis_name="s")


@functools.partial(
    pl.kernel,
    mesh=_MESH,
    out_type=jax.ShapeDtypeStruct((_L, _D, _B), jnp.float32),
    scratch_types=[
        pltpu.VMEM((4, 128), jnp.int32),
        pltpu.VMEM((4, 128), jnp.int32),
        pltpu.VMEM((4, 128), jnp.int32),
        pltpu.VMEM((4, 128), jnp.int32),
        pltpu.VMEM((_G, 2 * _D), jnp.float32),
        pltpu.VMEM((_G, 2 * _D), jnp.float32),
        pltpu.VMEM((_D, _G), jnp.float32),
        pltpu.VMEM((_D, _G), jnp.float32),
        pltpu.SemaphoreType.DMA,
        pltpu.SemaphoreType.DMA,
        pltpu.SemaphoreType.DMA,
        pltpu.SemaphoreType.DMA,
        pltpu.SemaphoreType.DMA,
    ],
    compiler_params=pltpu.CompilerParams(needs_layout_passes=False),
)
def _gather_kernel(table_hbm, idx_hbm, out_hbm,
                   idx0, idx1, rid0, rid1, rows0, rows1, tr0, tr1,
                   sg0, sg1, so0, so1, si):
    wid = lax.axis_index("s") * _NC + lax.axis_index("c")
    b0 = wid * _BW
    lanes = lax.iota(jnp.int32, 16)
    idxb = (idx0, idx1)
    ridb = (rid0, rid1)
    rowsb = (rows0, rows1)
    trb = (tr0, tr1)
    sgb = (sg0, sg1)
    sob = (so0, so1)

    def compute_rid(p):
        for k in range(4):
            for c in range(8):
                v = idxb[p][k, pl.ds(c * 16, 16)]
                ridb[p][k, pl.ds(c * 16, 16)] = jnp.where(v < _H, v, v - _H)

    def gather_copies(p, h):
        return [
            pltpu.make_async_copy(table_hbm.at[ridb[p].at[2 * h + k]],
                                  rowsb[h].at[pl.ds(k * 128, 128)], sgb[h])
            for k in range(_KSUB)
        ]

    def fire_gathers(p, h):
        for cp in gather_copies(p, h):
            cp.start()

    def wait_gathers(p, h):
        for cp in gather_copies(p, h):
            cp.wait()

    def transpose(p, h, l):
        @plsc.parallel_loop(0, _G // 16, unroll=8)
        def j_body(j):
            tok = idxb[p][2 * h + (j >> 3), pl.ds((j & 7) * 16, 16)]
            pcol = jnp.where(tok < _H, 0, _D)
            col_ids = j * 16 + lanes
            for k in range(16):
                rot = (lanes + k) & 15
                base = pcol + rot
                for d0 in range(0, _D, 16):
                    v = plsc.load_gather(rowsb[h], [col_ids, base + d0])
                    plsc.store_scatter(trb[h], [rot + d0, col_ids], v)

    def out_copy(h, l):
        return pltpu.make_async_copy(
            trb[h], out_hbm.at[l, :, pl.ds(b0 + h * _G, _G)], sob[h])

    pltpu.sync_copy(idx_hbm.at[0, wid], idx0)
    compute_rid(0)
    fire_gathers(0, 0)

    def pair_body(lp, carry):
        for ph in range(2):
            l = 2 * lp + ph
            q = 1 - ph
            @pl.when(l < _L - 1)
            def _():
                pltpu.make_async_copy(idx_hbm.at[l + 1, wid],
                                      idxb[q], si).start()
            fire_gathers(ph, 1)
            wait_gathers(ph, 0)

            @pl.when(l > 0)
            def _():
                out_copy(0, l - 1).wait()
            transpose(ph, 0, l)
            out_copy(0, l).start()

            @pl.when(l < _L - 1)
            def _():
                pltpu.make_async_copy(idx_hbm.at[l + 1, wid],
                                      idxb[q], si).wait()
                compute_rid(q)
            wait_gathers(ph, 1)

            @pl.when(l < _L - 1)
            def _():
                fire_gathers(q, 0)

            @pl.when(l > 0)
            def _():
                out_copy(1, l - 1).wait()
            transpose(ph, 1, l)
            out_copy(1, l).start()
        return carry

    lax.fori_loop(0, _L // 2, pair_body, 0)
    out_copy(0, _L - 1).wait()
    out_copy(1, _L - 1).wait()


def kernel(token_indices, lookup):
    idx4 = (token_indices.astype(jnp.int32).T
            .reshape(_L, _NW, _BW // 128, 128))
    table_t = _transpose_table(lookup)
    out3 = _gather_kernel(table_t, idx4)
    return jnp.transpose(out3, (1, 2, 0))


# speedup vs baseline: 1.1748x; 1.1748x over previous
import functools

import jax
import jax.numpy as jnp
from jax import lax
from jax.experimental import pallas as pl
from jax.experimental.pallas import tpu as pltpu
from jax.experimental.pallas import tpu_sc as plsc

_D = 64
_V = 1000000
_B = 16384
_L = 50
_NC, _NS = 2, 16
_NW = _NC * _NS
_BW = _B // _NW
_G = 256
_KSUB = _G // 128


_W = 16384
_GRID_T = 31
_H = _GRID_T * _W


def _tr_body(xlo_ref, xhi_ref, o_ref):
    o_ref[:, 0:_D] = xlo_ref[...].T
    o_ref[:, _D:2 * _D] = xhi_ref[...].T


def _transpose_table(lookup):
    return pl.pallas_call(
        _tr_body,
        grid=(_GRID_T,),
        in_specs=[
            pl.BlockSpec((_D, _W), lambda i: (0, i)),
            pl.BlockSpec((_D, _W), lambda i: (0, jnp.minimum(i + _GRID_T, 61))),
        ],
        out_specs=pl.BlockSpec((_W, 2 * _D), lambda i: (i, 0)),
        out_shape=jax.ShapeDtypeStruct((_H, 2 * _D), jnp.float32),
    )(lookup, lookup)


_MESH = plsc.VectorSubcoreMesh(core_axis_name="c", subcore_axis_name="s")


@functools.partial(
    pl.kernel,
    mesh=_MESH,
    out_type=jax.ShapeDtypeStruct((_L, _D, _B), jnp.float32),
    scratch_types=[
        pltpu.VMEM((4, 128), jnp.int32),
        pltpu.VMEM((4, 128), jnp.int32),
        pltpu.VMEM((4, 128), jnp.int32),
        pltpu.VMEM((4, 128), jnp.int32),
        pltpu.VMEM((_G, 2 * _D), jnp.float32),
        pltpu.VMEM((_G, 2 * _D), jnp.float32),
        pltpu.VMEM((_D, _G), jnp.float32),
        pltpu.VMEM((_D, _G), jnp.float32),
        pltpu.SemaphoreType.DMA,
        pltpu.SemaphoreType.DMA,
        pltpu.SemaphoreType.DMA,
        pltpu.SemaphoreType.DMA,
        pltpu.SemaphoreType.DMA,
    ],
    compiler_params=pltpu.CompilerParams(needs_layout_passes=False),
)
def _gather_kernel(table_hbm, idx_hbm, out_hbm,
                   idx0, idx1, rid0, rid1, rows0, rows1, tr0, tr1,
                   sg0, sg1, so0, so1, si):
    wid = lax.axis_index("s") * _NC + lax.axis_index("c")
    b0 = wid * _BW
    lanes = lax.iota(jnp.int32, 16)
    idxb = (idx0, idx1)
    ridb = (rid0, rid1)
    rowsb = (rows0, rows1)
    trb = (tr0, tr1)
    sgb = (sg0, sg1)
    sob = (so0, so1)

    def compute_rid(p):
        for k in range(4):
            for c in range(8):
                v = idxb[p][k, pl.ds(c * 16, 16)]
                ridb[p][k, pl.ds(c * 16, 16)] = jnp.where(v < _H, v, v - _H)

    def gather_copies(p, h):
        return [
            pltpu.make_async_copy(table_hbm.at[ridb[p].at[2 * h + k]],
                                  rowsb[h].at[pl.ds(k * 128, 128)], sgb[h])
            for k in range(_KSUB)
        ]

    def fire_gathers(p, h):
        for cp in gather_copies(p, h):
            cp.start()

    def wait_gathers(p, h):
        for cp in gather_copies(p, h):
            cp.wait()

    def transpose(p, h, l):
        @plsc.parallel_loop(0, _G // 16, unroll=4)
        def j_body(j):
            tok = idxb[p][2 * h + (j >> 3), pl.ds((j & 7) * 16, 16)]
            pcol = jnp.where(tok < _H, 0, _D)
            col_ids = j * 16 + lanes
            for k in range(16):
                rot = (lanes + k) & 15
                base = pcol + rot
                for d0 in range(0, _D, 16):
                    v = plsc.load_gather(rowsb[h], [col_ids, base + d0])
                    plsc.store_scatter(trb[h], [rot + d0, col_ids], v)

    def out_copy(h, l):
        return pltpu.make_async_copy(
            trb[h], out_hbm.at[l, :, pl.ds(b0 + h * _G, _G)], sob[h])

    pltpu.sync_copy(idx_hbm.at[0, wid], idx0)
    compute_rid(0)
    fire_gathers(0, 0)

    def pair_body(lp, carry):
        for ph in range(2):
            l = 2 * lp + ph
            q = 1 - ph
            @pl.when(l < _L - 1)
            def _():
                pltpu.make_async_copy(idx_hbm.at[l + 1, wid],
                                      idxb[q], si).start()
            fire_gathers(ph, 1)
            wait_gathers(ph, 0)

            @pl.when(l > 0)
            def _():
                out_copy(0, l - 1).wait()
            transpose(ph, 0, l)
            out_copy(0, l).start()

            @pl.when(l < _L - 1)
            def _():
                pltpu.make_async_copy(idx_hbm.at[l + 1, wid],
                                      idxb[q], si).wait()
                compute_rid(q)
            wait_gathers(ph, 1)

            @pl.when(l < _L - 1)
            def _():
                fire_gathers(q, 0)

            @pl.when(l > 0)
            def _():
                out_copy(1, l - 1).wait()
            transpose(ph, 1, l)
            out_copy(1, l).start()
        return carry

    lax.fori_loop(0, _L // 2, pair_body, 0)
    out_copy(0, _L - 1).wait()
    out_copy(1, _L - 1).wait()


def kernel(token_indices, lookup):
    idx4 = (token_indices.astype(jnp.int32).T
            .reshape(_L, _NW, _BW // 128, 128))
    table_t = _transpose_table(lookup)
    out3 = _gather_kernel(table_t, idx4)
    return jnp.transpose(out3, (1, 2, 0))
